# Initial kernel scaffold; baseline (speedup 1.0000x reference)
import jax
import jax.numpy as jnp
from jax.experimental import pallas as pl


def kernel(x, edge_index, gcn_W, gcn_b, bn_gamma, bn_beta, enc_W1, enc_b1, enc_Wmu, enc_bmu, enc_Wlv, enc_blv, dec_W1, dec_b1, dec_W2, dec_b2, eps):
    raise NotImplementedError("write your pallas kernel here")


# speedup vs baseline: 90.9973x; 90.9973x over previous
import functools

import jax
import jax.numpy as jnp
from jax import lax
from jax.experimental import pallas as pl
from jax.experimental.pallas import tpu as pltpu
from jax.experimental.pallas import tpu_sc as plsc

B, N, D = 32, 256, 128
E = 8192
INPUT_DIM = N * D
HIDDEN = 512
L = 128

KC = 2048
NC = 2048


def _adj_body(edges_hbm, zeros_hbm, out_hbm, ei_v, a_v):
    g = lax.axis_index("s") * 2 + lax.axis_index("c")
    pltpu.sync_copy(edges_hbm.at[g], ei_v)
    pltpu.sync_copy(zeros_hbm, a_v)
    ones = jnp.full((16,), 1.0, jnp.float32)

    def body(e, carry):
        src = ei_v[0, pl.ds(e * 16, 16)]
        dst = ei_v[1, pl.ds(e * 16, 16)]
        plsc.addupdate_scatter(a_v, [dst * N + src], ones)
        return carry

    lax.fori_loop(0, E // 16, body, 0)
    pltpu.sync_copy(a_v, out_hbm.at[g])


@functools.cache
def _build_adj_fn():
    return functools.partial(
        pl.kernel,
        out_type=jax.ShapeDtypeStruct((B, N * N), jnp.float32),
        mesh=plsc.VectorSubcoreMesh(core_axis_name="c", subcore_axis_name="s"),
        scratch_types=[
            pltpu.VMEM((2, E), jnp.int32),
            pltpu.VMEM((N * N,), jnp.float32),
        ],
        compiler_params=pltpu.CompilerParams(needs_layout_passes=False),
    )(_adj_body)


def _gcn_body(a_ref, x_ref, w_ref, b_ref, gam_ref, bet_ref, o_ref):
    A = a_ref[0]
    xg = x_ref[0]
    h = jnp.dot(xg, w_ref[...], preferred_element_type=jnp.float32)
    deg = jnp.sum(A, axis=1) + 1.0
    dinv = lax.rsqrt(deg)
    hs = h * dinv[:, None]
    out = jnp.dot(A, hs, preferred_element_type=jnp.float32) + hs
    out = out * dinv[:, None] + b_ref[...]
    mu = jnp.mean(out, axis=0, keepdims=True)
    var = jnp.mean((out - mu) * (out - mu), axis=0, keepdims=True)
    out = (out - mu) * lax.rsqrt(var + 1e-5) * gam_ref[...] + bet_ref[...]
    o_ref[0] = jnp.tanh(out)


def _gcn_dense(A3, x, gcn_W, gcn_b, bn_gamma, bn_beta):
    return pl.pallas_call(
        _gcn_body,
        grid=(B,),
        in_specs=[
            pl.BlockSpec((1, N, N), lambda b: (b, 0, 0)),
            pl.BlockSpec((1, N, D), lambda b: (b, 0, 0)),
            pl.BlockSpec((D, D), lambda b: (0, 0)),
            pl.BlockSpec((1, D), lambda b: (0, 0)),
            pl.BlockSpec((1, D), lambda b: (0, 0)),
            pl.BlockSpec((1, D), lambda b: (0, 0)),
        ],
        out_specs=pl.BlockSpec((1, N, D), lambda b: (b, 0, 0)),
        out_shape=jax.ShapeDtypeStruct((B, N, D), jnp.float32),
        compiler_params=pltpu.CompilerParams(
            dimension_semantics=("parallel",),
        ),
    )(A3, x, gcn_W, gcn_b.reshape(1, D), bn_gamma.reshape(1, D),
      bn_beta.reshape(1, D))


def _enc_body(xs_ref, w1_ref, b1_ref, wmu_ref, bmu_ref, wlv_ref, blv_ref,
              dw1_ref, db1_ref, eps_ref, mean_ref, lv_ref, hd_ref, acc_ref):
    k = pl.program_id(0)

    @pl.when(k == 0)
    def _():
        acc_ref[...] = jnp.zeros_like(acc_ref)

    acc_ref[...] += jnp.dot(xs_ref[...], w1_ref[...],
                            preferred_element_type=jnp.float32)

    @pl.when(k == pl.num_programs(0) - 1)
    def _():
        h1 = jnp.maximum(acc_ref[...] + b1_ref[...], 0.0)
        mean = jnp.dot(h1, wmu_ref[...],
                       preferred_element_type=jnp.float32) + bmu_ref[...]
        lv = jnp.dot(h1, wlv_ref[...],
                     preferred_element_type=jnp.float32) + blv_ref[...]
        z = mean + jnp.exp(0.5 * lv) * eps_ref[...]
        hd = jnp.dot(z, dw1_ref[...],
                     preferred_element_type=jnp.float32) + db1_ref[...]
        mean_ref[...] = mean
        lv_ref[...] = lv
        hd_ref[...] = jnp.maximum(hd, 0.0)


def _encoder(xs, enc_W1, enc_b1, enc_Wmu, enc_bmu, enc_Wlv, enc_blv,
             dec_W1, dec_b1, eps):
    nsteps = INPUT_DIM // KC
    return pl.pallas_call(
        _enc_body,
        grid=(nsteps,),
        in_specs=[
            pl.BlockSpec((B, KC), lambda k: (0, k)),
            pl.BlockSpec((KC, HIDDEN), lambda k: (k, 0)),
            pl.BlockSpec((1, HIDDEN), lambda k: (0, 0)),
            pl.BlockSpec((HIDDEN, L), lambda k: (0, 0)),
            pl.BlockSpec((1, L), lambda k: (0, 0)),
            pl.BlockSpec((HIDDEN, L), lambda k: (0, 0)),
            pl.BlockSpec((1, L), lambda k: (0, 0)),
            pl.BlockSpec((L, HIDDEN), lambda k: (0, 0)),
            pl.BlockSpec((1, HIDDEN), lambda k: (0, 0)),
            pl.BlockSpec((B, L), lambda k: (0, 0)),
        ],
        out_specs=[
            pl.BlockSpec((B, L), lambda k: (0, 0)),
            pl.BlockSpec((B, L), lambda k: (0, 0)),
            pl.BlockSpec((B, HIDDEN), lambda k: (0, 0)),
        ],
        out_shape=[
            jax.ShapeDtypeStruct((B, L), jnp.float32),
            jax.ShapeDtypeStruct((B, L), jnp.float32),
            jax.ShapeDtypeStruct((B, HIDDEN), jnp.float32),
        ],
        scratch_shapes=[pltpu.VMEM((B, HIDDEN), jnp.float32)],
        compiler_params=pltpu.CompilerParams(
            dimension_semantics=("arbitrary",),
        ),
    )(xs, enc_W1, enc_b1.reshape(1, HIDDEN), enc_Wmu,
      enc_bmu.reshape(1, L), enc_Wlv, enc_blv.reshape(1, L),
      dec_W1, dec_b1.reshape(1, HIDDEN), eps)


def _dec_body(hd_ref, w2_ref, b2_ref, o_ref):
    o_ref[...] = jax.nn.sigmoid(
        jnp.dot(hd_ref[...], w2_ref[...],
                preferred_element_type=jnp.float32) + b2_ref[...])


def _decoder(hd, dec_W2, dec_b2):
    nsteps = INPUT_DIM // NC
    return pl.pallas_call(
        _dec_body,
        grid=(nsteps,),
        in_specs=[
            pl.BlockSpec((B, HIDDEN), lambda n: (0, 0)),
            pl.BlockSpec((HIDDEN, NC), lambda n: (0, n)),
            pl.BlockSpec((1, NC), lambda n: (0, n)),
        ],
        out_specs=pl.BlockSpec((B, NC), lambda n: (0, n)),
        out_shape=jax.ShapeDtypeStruct((B, INPUT_DIM), jnp.float32),
        compiler_params=pltpu.CompilerParams(
            dimension_semantics=("parallel",),
        ),
    )(hd, dec_W2, dec_b2.reshape(1, INPUT_DIM))


def kernel(x, edge_index, gcn_W, gcn_b, bn_gamma, bn_beta,
           enc_W1, enc_b1, enc_Wmu, enc_bmu, enc_Wlv, enc_blv,
           dec_W1, dec_b1, dec_W2, dec_b2, eps):
    zeros = jnp.zeros((N * N,), jnp.float32)
    A3 = _build_adj_fn()(edge_index, zeros).reshape(B, N, N)
    xs = _gcn_dense(A3, x, gcn_W, gcn_b, bn_gamma, bn_beta)
    xs = xs.reshape(B, INPUT_DIM)
    mean, log_var, hd = _encoder(xs, enc_W1, enc_b1, enc_Wmu, enc_bmu,
                                 enc_Wlv, enc_blv, dec_W1, dec_b1, eps)
    x_hat = _decoder(hd, dec_W2, dec_b2)
    return (x_hat, mean, log_var)


# speedup vs baseline: 93.2050x; 1.0243x over previous
import functools

import jax
import jax.numpy as jnp
from jax import lax
from jax.experimental import pallas as pl
from jax.experimental.pallas import tpu as pltpu
from jax.experimental.pallas import tpu_sc as plsc

B, N, D = 32, 256, 128
E = 8192
INPUT_DIM = N * D
HIDDEN = 512
L = 128

KC = 2048
NC = 2048


def _adj_body(edges_hbm, zeros_hbm, out_hbm, ei_v, a_v):
    g = lax.axis_index("s") * 2 + lax.axis_index("c")
    pltpu.sync_copy(edges_hbm.at[g], ei_v)
    pltpu.sync_copy(zeros_hbm, a_v)
    ones = jnp.full((16,), 1.0, jnp.float32)

    def body(e, carry):
        src = ei_v[0, pl.ds(e * 16, 16)]
        dst = ei_v[1, pl.ds(e * 16, 16)]
        plsc.addupdate_scatter(a_v, [dst * N + src], ones)
        return carry

    lax.fori_loop(0, E // 16, body, 0)
    pltpu.sync_copy(a_v, out_hbm.at[g])


@functools.cache
def _build_adj_fn():
    return functools.partial(
        pl.kernel,
        out_type=jax.ShapeDtypeStruct((B, N * N), jnp.float32),
        mesh=plsc.VectorSubcoreMesh(core_axis_name="c", subcore_axis_name="s"),
        scratch_types=[
            pltpu.VMEM((2, E), jnp.int32),
            pltpu.VMEM((N * N,), jnp.float32),
        ],
        compiler_params=pltpu.CompilerParams(needs_layout_passes=False),
    )(_adj_body)


def _gcn_body(a_ref, x_ref, w_ref, b_ref, gam_ref, bet_ref, o_ref):
    A = a_ref[0]
    xg = x_ref[0]
    h = jnp.dot(xg, w_ref[...], preferred_element_type=jnp.float32)
    deg = jnp.sum(A, axis=1) + 1.0
    dinv = lax.rsqrt(deg)
    hs = h * dinv[:, None]
    out = jnp.dot(A, hs, preferred_element_type=jnp.float32) + hs
    out = out * dinv[:, None] + b_ref[...]
    mu = jnp.mean(out, axis=0, keepdims=True)
    var = jnp.mean((out - mu) * (out - mu), axis=0, keepdims=True)
    out = (out - mu) * lax.rsqrt(var + 1e-5) * gam_ref[...] + bet_ref[...]
    o_ref[0] = jnp.tanh(out).astype(jnp.bfloat16)


def _gcn_dense(A3, x, gcn_W, gcn_b, bn_gamma, bn_beta):
    return pl.pallas_call(
        _gcn_body,
        grid=(B,),
        in_specs=[
            pl.BlockSpec((1, N, N), lambda b: (b, 0, 0)),
            pl.BlockSpec((1, N, D), lambda b: (b, 0, 0)),
            pl.BlockSpec((D, D), lambda b: (0, 0)),
            pl.BlockSpec((1, D), lambda b: (0, 0)),
            pl.BlockSpec((1, D), lambda b: (0, 0)),
            pl.BlockSpec((1, D), lambda b: (0, 0)),
        ],
        out_specs=pl.BlockSpec((1, N, D), lambda b: (b, 0, 0)),
        out_shape=jax.ShapeDtypeStruct((B, N, D), jnp.bfloat16),
        compiler_params=pltpu.CompilerParams(
            dimension_semantics=("parallel",),
        ),
    )(A3, x, gcn_W, gcn_b.reshape(1, D), bn_gamma.reshape(1, D),
      bn_beta.reshape(1, D))


def _enc_body(xs_ref, w1_ref, b1_ref, wmu_ref, bmu_ref, wlv_ref, blv_ref,
              dw1_ref, db1_ref, eps_ref, mean_ref, lv_ref, hd_ref, acc_ref):
    k = pl.program_id(0)

    @pl.when(k == 0)
    def _():
        acc_ref[...] = jnp.zeros_like(acc_ref)

    acc_ref[...] += jnp.dot(xs_ref[...], w1_ref[...],
                            preferred_element_type=jnp.float32)

    @pl.when(k == pl.num_programs(0) - 1)
    def _():
        h1 = jnp.maximum(acc_ref[...] + b1_ref[...], 0.0)
        mean = jnp.dot(h1, wmu_ref[...],
                       preferred_element_type=jnp.float32) + bmu_ref[...]
        lv = jnp.dot(h1, wlv_ref[...],
                     preferred_element_type=jnp.float32) + blv_ref[...]
        z = mean + jnp.exp(0.5 * lv) * eps_ref[...]
        hd = jnp.dot(z, dw1_ref[...],
                     preferred_element_type=jnp.float32) + db1_ref[...]
        mean_ref[...] = mean
        lv_ref[...] = lv
        hd_ref[...] = jnp.maximum(hd, 0.0).astype(jnp.bfloat16)


def _encoder(xs, enc_W1, enc_b1, enc_Wmu, enc_bmu, enc_Wlv, enc_blv,
             dec_W1, dec_b1, eps):
    nsteps = INPUT_DIM // KC
    return pl.pallas_call(
        _enc_body,
        grid=(nsteps,),
        in_specs=[
            pl.BlockSpec((B, KC), lambda k: (0, k)),
            pl.BlockSpec((KC, HIDDEN), lambda k: (k, 0)),
            pl.BlockSpec((1, HIDDEN), lambda k: (0, 0)),
            pl.BlockSpec((HIDDEN, L), lambda k: (0, 0)),
            pl.BlockSpec((1, L), lambda k: (0, 0)),
            pl.BlockSpec((HIDDEN, L), lambda k: (0, 0)),
            pl.BlockSpec((1, L), lambda k: (0, 0)),
            pl.BlockSpec((L, HIDDEN), lambda k: (0, 0)),
            pl.BlockSpec((1, HIDDEN), lambda k: (0, 0)),
            pl.BlockSpec((B, L), lambda k: (0, 0)),
        ],
        out_specs=[
            pl.BlockSpec((B, L), lambda k: (0, 0)),
            pl.BlockSpec((B, L), lambda k: (0, 0)),
            pl.BlockSpec((B, HIDDEN), lambda k: (0, 0)),
        ],
        out_shape=[
            jax.ShapeDtypeStruct((B, L), jnp.float32),
            jax.ShapeDtypeStruct((B, L), jnp.float32),
            jax.ShapeDtypeStruct((B, HIDDEN), jnp.bfloat16),
        ],
        scratch_shapes=[pltpu.VMEM((B, HIDDEN), jnp.float32)],
        compiler_params=pltpu.CompilerParams(
            dimension_semantics=("arbitrary",),
        ),
    )(xs, enc_W1, enc_b1.reshape(1, HIDDEN), enc_Wmu,
      enc_bmu.reshape(1, L), enc_Wlv, enc_blv.reshape(1, L),
      dec_W1, dec_b1.reshape(1, HIDDEN), eps)


def _dec_body(hd_ref, w2_ref, b2_ref, o_ref):
    o_ref[...] = jax.nn.sigmoid(
        jnp.dot(hd_ref[...], w2_ref[...],
                preferred_element_type=jnp.float32) + b2_ref[...])


def _decoder(hd, dec_W2, dec_b2):
    nsteps = INPUT_DIM // NC
    return pl.pallas_call(
        _dec_body,
        grid=(nsteps,),
        in_specs=[
            pl.BlockSpec((B, HIDDEN), lambda n: (0, 0)),
            pl.BlockSpec((HIDDEN, NC), lambda n: (0, n)),
            pl.BlockSpec((1, NC), lambda n: (0, n)),
        ],
        out_specs=pl.BlockSpec((B, NC), lambda n: (0, n)),
        out_shape=jax.ShapeDtypeStruct((B, INPUT_DIM), jnp.float32),
        compiler_params=pltpu.CompilerParams(
            dimension_semantics=("parallel",),
        ),
    )(hd, dec_W2, dec_b2.reshape(1, INPUT_DIM))


def kernel(x, edge_index, gcn_W, gcn_b, bn_gamma, bn_beta,
           enc_W1, enc_b1, enc_Wmu, enc_bmu, enc_Wlv, enc_blv,
           dec_W1, dec_b1, dec_W2, dec_b2, eps):
    zeros = jnp.zeros((N * N,), jnp.float32)
    A3 = _build_adj_fn()(edge_index, zeros).reshape(B, N, N)
    xs = _gcn_dense(A3, x, gcn_W, gcn_b, bn_gamma, bn_beta)
    xs = xs.reshape(B, INPUT_DIM)
    mean, log_var, hd = _encoder(xs, enc_W1, enc_b1,
                                 enc_Wmu, enc_bmu, enc_Wlv, enc_blv,
                                 dec_W1, dec_b1, eps)
    x_hat = _decoder(hd, dec_W2, dec_b2)
    return (x_hat, mean, log_var)


# speedup vs baseline: 97.0642x; 1.0414x over previous
import functools

import jax
import jax.numpy as jnp
from jax import lax
from jax.experimental import pallas as pl
from jax.experimental.pallas import tpu as pltpu
from jax.experimental.pallas import tpu_sc as plsc

B, N, D = 32, 256, 128
E = 8192
INPUT_DIM = N * D
HIDDEN = 512
L = 128

KC = 2048
NC = 2048


def _adj_body(edges_hbm, out_hbm, ei_v, a_v, sem):
    g = lax.axis_index("s") * 2 + lax.axis_index("c")
    cp = pltpu.async_copy(edges_hbm.at[g], ei_v, sem)
    zero = jnp.zeros((16,), jnp.float32)
    ones = jnp.full((16,), 1.0, jnp.float32)

    def zbody(i, carry):
        base = i * 64
        a_v[pl.ds(base, 16)] = zero
        a_v[pl.ds(base + 16, 16)] = zero
        a_v[pl.ds(base + 32, 16)] = zero
        a_v[pl.ds(base + 48, 16)] = zero
        return carry

    lax.fori_loop(0, N * N // 64, zbody, 0)
    cp.wait()

    def body(e, carry):
        base = e * 64
        for j in range(4):
            src = ei_v[0, pl.ds(base + j * 16, 16)]
            dst = ei_v[1, pl.ds(base + j * 16, 16)]
            plsc.addupdate_scatter(a_v, [dst * N + src], ones)
        return carry

    lax.fori_loop(0, E // 64, body, 0)
    pltpu.sync_copy(a_v, out_hbm.at[g])


@functools.cache
def _build_adj_fn():
    return functools.partial(
        pl.kernel,
        out_type=jax.ShapeDtypeStruct((B, N * N), jnp.float32),
        mesh=plsc.VectorSubcoreMesh(core_axis_name="c", subcore_axis_name="s"),
        scratch_types=[
            pltpu.VMEM((2, E), jnp.int32),
            pltpu.VMEM((N * N,), jnp.float32),
            pltpu.SemaphoreType.DMA,
        ],
        compiler_params=pltpu.CompilerParams(needs_layout_passes=False),
    )(_adj_body)


def _gcn_body(a_ref, x_ref, w_ref, b_ref, gam_ref, bet_ref, o_ref):
    A = a_ref[0]
    xg = x_ref[0]
    h = jnp.dot(xg, w_ref[...], preferred_element_type=jnp.float32)
    deg = jnp.sum(A, axis=1) + 1.0
    dinv = lax.rsqrt(deg)
    hs = h * dinv[:, None]
    out = jnp.dot(A, hs, preferred_element_type=jnp.float32) + hs
    out = out * dinv[:, None] + b_ref[...]
    mu = jnp.mean(out, axis=0, keepdims=True)
    var = jnp.mean((out - mu) * (out - mu), axis=0, keepdims=True)
    out = (out - mu) * lax.rsqrt(var + 1e-5) * gam_ref[...] + bet_ref[...]
    o_ref[0] = jnp.tanh(out).astype(jnp.bfloat16)


def _gcn_dense(A3, x, gcn_W, gcn_b, bn_gamma, bn_beta):
    return pl.pallas_call(
        _gcn_body,
        grid=(B,),
        in_specs=[
            pl.BlockSpec((1, N, N), lambda b: (b, 0, 0)),
            pl.BlockSpec((1, N, D), lambda b: (b, 0, 0)),
            pl.BlockSpec((D, D), lambda b: (0, 0)),
            pl.BlockSpec((1, D), lambda b: (0, 0)),
            pl.BlockSpec((1, D), lambda b: (0, 0)),
            pl.BlockSpec((1, D), lambda b: (0, 0)),
        ],
        out_specs=pl.BlockSpec((1, N, D), lambda b: (b, 0, 0)),
        out_shape=jax.ShapeDtypeStruct((B, N, D), jnp.bfloat16),
        compiler_params=pltpu.CompilerParams(
            dimension_semantics=("parallel",),
        ),
    )(A3, x, gcn_W, gcn_b.reshape(1, D), bn_gamma.reshape(1, D),
      bn_beta.reshape(1, D))


def _enc_body(xs_ref, w1_ref, b1_ref, wmu_ref, bmu_ref, wlv_ref, blv_ref,
              dw1_ref, db1_ref, eps_ref, mean_ref, lv_ref, hd_ref, acc_ref):
    k = pl.program_id(0)

    @pl.when(k == 0)
    def _():
        acc_ref[...] = jnp.zeros_like(acc_ref)

    acc_ref[...] += jnp.dot(xs_ref[...], w1_ref[...],
                            preferred_element_type=jnp.float32)

    @pl.when(k == pl.num_programs(0) - 1)
    def _():
        h1 = jnp.maximum(acc_ref[...] + b1_ref[...], 0.0)
        mean = jnp.dot(h1, wmu_ref[...],
                       preferred_element_type=jnp.float32) + bmu_ref[...]
        lv = jnp.dot(h1, wlv_ref[...],
                     preferred_element_type=jnp.float32) + blv_ref[...]
        z = mean + jnp.exp(0.5 * lv) * eps_ref[...]
        hd = jnp.dot(z, dw1_ref[...],
                     preferred_element_type=jnp.float32) + db1_ref[...]
        mean_ref[...] = mean
        lv_ref[...] = lv
        hd_ref[...] = jnp.maximum(hd, 0.0).astype(jnp.bfloat16)


def _encoder(xs, enc_W1, enc_b1, enc_Wmu, enc_bmu, enc_Wlv, enc_blv,
             dec_W1, dec_b1, eps):
    nsteps = INPUT_DIM // KC
    return pl.pallas_call(
        _enc_body,
        grid=(nsteps,),
        in_specs=[
            pl.BlockSpec((B, KC), lambda k: (0, k)),
            pl.BlockSpec((KC, HIDDEN), lambda k: (k, 0)),
            pl.BlockSpec((1, HIDDEN), lambda k: (0, 0)),
            pl.BlockSpec((HIDDEN, L), lambda k: (0, 0)),
            pl.BlockSpec((1, L), lambda k: (0, 0)),
            pl.BlockSpec((HIDDEN, L), lambda k: (0, 0)),
            pl.BlockSpec((1, L), lambda k: (0, 0)),
            pl.BlockSpec((L, HIDDEN), lambda k: (0, 0)),
            pl.BlockSpec((1, HIDDEN), lambda k: (0, 0)),
            pl.BlockSpec((B, L), lambda k: (0, 0)),
        ],
        out_specs=[
            pl.BlockSpec((B, L), lambda k: (0, 0)),
            pl.BlockSpec((B, L), lambda k: (0, 0)),
            pl.BlockSpec((B, HIDDEN), lambda k: (0, 0)),
        ],
        out_shape=[
            jax.ShapeDtypeStruct((B, L), jnp.float32),
            jax.ShapeDtypeStruct((B, L), jnp.float32),
            jax.ShapeDtypeStruct((B, HIDDEN), jnp.bfloat16),
        ],
        scratch_shapes=[pltpu.VMEM((B, HIDDEN), jnp.float32)],
        compiler_params=pltpu.CompilerParams(
            dimension_semantics=("arbitrary",),
        ),
    )(xs, enc_W1, enc_b1.reshape(1, HIDDEN), enc_Wmu,
      enc_bmu.reshape(1, L), enc_Wlv, enc_blv.reshape(1, L),
      dec_W1, dec_b1.reshape(1, HIDDEN), eps)


def _dec_body(hd_ref, w2_ref, b2_ref, o_ref):
    o_ref[...] = jax.nn.sigmoid(
        jnp.dot(hd_ref[...], w2_ref[...],
                preferred_element_type=jnp.float32) + b2_ref[...])


def _decoder(hd, dec_W2, dec_b2):
    nsteps = INPUT_DIM // NC
    return pl.pallas_call(
        _dec_body,
        grid=(nsteps,),
        in_specs=[
            pl.BlockSpec((B, HIDDEN), lambda n: (0, 0)),
            pl.BlockSpec((HIDDEN, NC), lambda n: (0, n)),
            pl.BlockSpec((1, NC), lambda n: (0, n)),
        ],
        out_specs=pl.BlockSpec((B, NC), lambda n: (0, n)),
        out_shape=jax.ShapeDtypeStruct((B, INPUT_DIM), jnp.float32),
        compiler_params=pltpu.CompilerParams(
            dimension_semantics=("parallel",),
        ),
    )(hd, dec_W2, dec_b2.reshape(1, INPUT_DIM))


def kernel(x, edge_index, gcn_W, gcn_b, bn_gamma, bn_beta,
           enc_W1, enc_b1, enc_Wmu, enc_bmu, enc_Wlv, enc_blv,
           dec_W1, dec_b1, dec_W2, dec_b2, eps):
    A3 = _build_adj_fn()(edge_index).reshape(B, N, N)
    xs = _gcn_dense(A3, x, gcn_W, gcn_b, bn_gamma, bn_beta)
    xs = xs.reshape(B, INPUT_DIM)
    mean, log_var, hd = _encoder(xs, enc_W1, enc_b1,
                                 enc_Wmu, enc_bmu, enc_Wlv, enc_blv,
                                 dec_W1, dec_b1, eps)
    x_hat = _decoder(hd, dec_W2, dec_b2)
    return (x_hat, mean, log_var)


# speedup vs baseline: 113.4804x; 1.1691x over previous
import functools

import jax
import jax.numpy as jnp
from jax import lax
from jax.experimental import pallas as pl
from jax.experimental.pallas import tpu as pltpu
from jax.experimental.pallas import tpu_sc as plsc

B, N, D = 32, 256, 128
E = 8192
INPUT_DIM = N * D
HIDDEN = 512
L = 128

KC = 2048
NC = 2048


def _adj_body(edges_hbm, out_hbm, ei_v, a_v, sem):
    g = lax.axis_index("s") * 2 + lax.axis_index("c")
    cp = pltpu.async_copy(edges_hbm.at[g], ei_v, sem)
    zero = jnp.zeros((16,), jnp.float32)
    ones = jnp.full((16,), 1.0, jnp.float32)

    def zbody(i, carry):
        base = i * 64
        a_v[pl.ds(base, 16)] = zero
        a_v[pl.ds(base + 16, 16)] = zero
        a_v[pl.ds(base + 32, 16)] = zero
        a_v[pl.ds(base + 48, 16)] = zero
        return carry

    lax.fori_loop(0, N * N // 64, zbody, 0)
    cp.wait()

    def body(e, carry):
        base = e * 64
        for j in range(4):
            src = ei_v[0, pl.ds(base + j * 16, 16)]
            dst = ei_v[1, pl.ds(base + j * 16, 16)]
            plsc.addupdate_scatter(a_v, [dst * N + src], ones)
        return carry

    lax.fori_loop(0, E // 64, body, 0)
    pltpu.sync_copy(a_v, out_hbm.at[g])


@functools.cache
def _build_adj_fn():
    return functools.partial(
        pl.kernel,
        out_type=jax.ShapeDtypeStruct((B, N * N), jnp.float32),
        mesh=plsc.VectorSubcoreMesh(core_axis_name="c", subcore_axis_name="s"),
        scratch_types=[
            pltpu.VMEM((2, E), jnp.int32),
            pltpu.VMEM((N * N,), jnp.float32),
            pltpu.SemaphoreType.DMA,
        ],
        compiler_params=pltpu.CompilerParams(needs_layout_passes=False),
    )(_adj_body)


GB = 8


def _gcn_body(a_ref, x_ref, w_ref, b_ref, gam_ref, bet_ref, o_ref):
    A = a_ref[...]
    xg = x_ref[...]
    w = w_ref[...]
    h = lax.dot_general(xg, w, (((2,), (0,)), ((), ())),
                        preferred_element_type=jnp.float32)
    deg = jnp.sum(A, axis=2) + 1.0
    dinv = lax.rsqrt(deg)
    hs = h * dinv[:, :, None]
    out = lax.dot_general(A, hs, (((2,), (1,)), ((0,), (0,))),
                          preferred_element_type=jnp.float32) + hs
    out = out * dinv[:, :, None] + b_ref[...]
    mu = jnp.mean(out, axis=1, keepdims=True)
    var = jnp.mean((out - mu) * (out - mu), axis=1, keepdims=True)
    out = (out - mu) * lax.rsqrt(var + 1e-5) * gam_ref[...] + bet_ref[...]
    o_ref[...] = jnp.tanh(out).astype(jnp.bfloat16)


def _gcn_dense(A3, x, gcn_W, gcn_b, bn_gamma, bn_beta):
    return pl.pallas_call(
        _gcn_body,
        grid=(B // GB,),
        in_specs=[
            pl.BlockSpec((GB, N, N), lambda b: (b, 0, 0)),
            pl.BlockSpec((GB, N, D), lambda b: (b, 0, 0)),
            pl.BlockSpec((D, D), lambda b: (0, 0)),
            pl.BlockSpec((1, 1, D), lambda b: (0, 0, 0)),
            pl.BlockSpec((1, 1, D), lambda b: (0, 0, 0)),
            pl.BlockSpec((1, 1, D), lambda b: (0, 0, 0)),
        ],
        out_specs=pl.BlockSpec((GB, N, D), lambda b: (b, 0, 0)),
        out_shape=jax.ShapeDtypeStruct((B, N, D), jnp.bfloat16),
        compiler_params=pltpu.CompilerParams(
            dimension_semantics=("parallel",),
        ),
    )(A3, x, gcn_W, gcn_b.reshape(1, 1, D), bn_gamma.reshape(1, 1, D),
      bn_beta.reshape(1, 1, D))


def _enc_body(xs_ref, w1_ref, b1_ref, wmu_ref, bmu_ref, wlv_ref, blv_ref,
              dw1_ref, db1_ref, eps_ref, mean_ref, lv_ref, hd_ref, acc_ref):
    k = pl.program_id(0)

    @pl.when(k == 0)
    def _():
        acc_ref[...] = jnp.zeros_like(acc_ref)

    acc_ref[...] += jnp.dot(xs_ref[...], w1_ref[...],
                            preferred_element_type=jnp.float32)

    @pl.when(k == pl.num_programs(0) - 1)
    def _():
        h1 = jnp.maximum(acc_ref[...] + b1_ref[...], 0.0)
        mean = jnp.dot(h1, wmu_ref[...],
                       preferred_element_type=jnp.float32) + bmu_ref[...]
        lv = jnp.dot(h1, wlv_ref[...],
                     preferred_element_type=jnp.float32) + blv_ref[...]
        z = mean + jnp.exp(0.5 * lv) * eps_ref[...]
        hd = jnp.dot(z, dw1_ref[...],
                     preferred_element_type=jnp.float32) + db1_ref[...]
        mean_ref[...] = mean
        lv_ref[...] = lv
        hd_ref[...] = jnp.maximum(hd, 0.0).astype(jnp.bfloat16)


def _encoder(xs, enc_W1, enc_b1, enc_Wmu, enc_bmu, enc_Wlv, enc_blv,
             dec_W1, dec_b1, eps):
    nsteps = INPUT_DIM // KC
    return pl.pallas_call(
        _enc_body,
        grid=(nsteps,),
        in_specs=[
            pl.BlockSpec((B, KC), lambda k: (0, k)),
            pl.BlockSpec((KC, HIDDEN), lambda k: (k, 0)),
            pl.BlockSpec((1, HIDDEN), lambda k: (0, 0)),
            pl.BlockSpec((HIDDEN, L), lambda k: (0, 0)),
            pl.BlockSpec((1, L), lambda k: (0, 0)),
            pl.BlockSpec((HIDDEN, L), lambda k: (0, 0)),
            pl.BlockSpec((1, L), lambda k: (0, 0)),
            pl.BlockSpec((L, HIDDEN), lambda k: (0, 0)),
            pl.BlockSpec((1, HIDDEN), lambda k: (0, 0)),
            pl.BlockSpec((B, L), lambda k: (0, 0)),
        ],
        out_specs=[
            pl.BlockSpec((B, L), lambda k: (0, 0)),
            pl.BlockSpec((B, L), lambda k: (0, 0)),
            pl.BlockSpec((B, HIDDEN), lambda k: (0, 0)),
        ],
        out_shape=[
            jax.ShapeDtypeStruct((B, L), jnp.float32),
            jax.ShapeDtypeStruct((B, L), jnp.float32),
            jax.ShapeDtypeStruct((B, HIDDEN), jnp.bfloat16),
        ],
        scratch_shapes=[pltpu.VMEM((B, HIDDEN), jnp.float32)],
        compiler_params=pltpu.CompilerParams(
            dimension_semantics=("arbitrary",),
        ),
    )(xs, enc_W1, enc_b1.reshape(1, HIDDEN), enc_Wmu,
      enc_bmu.reshape(1, L), enc_Wlv, enc_blv.reshape(1, L),
      dec_W1, dec_b1.reshape(1, HIDDEN), eps)


def _dec_body(hd_ref, w2_ref, b2_ref, o_ref):
    o_ref[...] = jax.nn.sigmoid(
        jnp.dot(hd_ref[...], w2_ref[...],
                preferred_element_type=jnp.float32) + b2_ref[...])


def _decoder(hd, dec_W2, dec_b2):
    nsteps = INPUT_DIM // NC
    return pl.pallas_call(
        _dec_body,
        grid=(nsteps,),
        in_specs=[
            pl.BlockSpec((B, HIDDEN), lambda n: (0, 0)),
            pl.BlockSpec((HIDDEN, NC), lambda n: (0, n)),
            pl.BlockSpec((1, NC), lambda n: (0, n)),
        ],
        out_specs=pl.BlockSpec((B, NC), lambda n: (0, n)),
        out_shape=jax.ShapeDtypeStruct((B, INPUT_DIM), jnp.float32),
        compiler_params=pltpu.CompilerParams(
            dimension_semantics=("parallel",),
        ),
    )(hd, dec_W2, dec_b2.reshape(1, INPUT_DIM))


def kernel(x, edge_index, gcn_W, gcn_b, bn_gamma, bn_beta,
           enc_W1, enc_b1, enc_Wmu, enc_bmu, enc_Wlv, enc_blv,
           dec_W1, dec_b1, dec_W2, dec_b2, eps):
    A3 = _build_adj_fn()(edge_index).reshape(B, N, N)
    xs = _gcn_dense(A3, x, gcn_W, gcn_b, bn_gamma, bn_beta)
    xs = xs.reshape(B, INPUT_DIM)
    mean, log_var, hd = _encoder(xs, enc_W1, enc_b1,
                                 enc_Wmu, enc_bmu, enc_Wlv, enc_blv,
                                 dec_W1, dec_b1, eps)
    x_hat = _decoder(hd, dec_W2, dec_b2)
    return (x_hat, mean, log_var)


# speedup vs baseline: 126.5615x; 1.1153x over previous
import functools

import jax
import jax.numpy as jnp
from jax import lax
from jax.experimental import pallas as pl
from jax.experimental.pallas import tpu as pltpu
from jax.experimental.pallas import tpu_sc as plsc

B, N, D = 32, 256, 128
E = 8192
INPUT_DIM = N * D
HIDDEN = 512
L = 128

KC = 2048
NC = 2048


def _adj_body(edges_hbm, out_hbm, ei_v, a_v, sem):
    g = lax.axis_index("s") * 2 + lax.axis_index("c")
    cp = pltpu.async_copy(edges_hbm.at[g], ei_v, sem)
    zero = jnp.zeros((16,), jnp.float32)
    ones = jnp.full((16,), 1.0, jnp.float32)

    def zbody(r, carry):
        for j in range(N // 16):
            a_v[r, pl.ds(j * 16, 16)] = zero
        return carry

    lax.fori_loop(0, N, zbody, 0)
    cp.wait()

    def body(e, carry):
        base = e * 64
        for j in range(4):
            src = ei_v[0, pl.ds(base + j * 16, 16)]
            dst = ei_v[1, pl.ds(base + j * 16, 16)]
            plsc.addupdate_scatter(a_v, [dst, src], ones)
        return carry

    lax.fori_loop(0, E // 64, body, 0)
    pltpu.sync_copy(a_v, out_hbm.at[g])


@functools.cache
def _build_adj_fn():
    return functools.partial(
        pl.kernel,
        out_type=jax.ShapeDtypeStruct((B, N, N), jnp.float32),
        mesh=plsc.VectorSubcoreMesh(core_axis_name="c", subcore_axis_name="s"),
        scratch_types=[
            pltpu.VMEM((2, E), jnp.int32),
            pltpu.VMEM((N, N), jnp.float32),
            pltpu.SemaphoreType.DMA,
        ],
        compiler_params=pltpu.CompilerParams(needs_layout_passes=False),
    )(_adj_body)


GB = 8


def _gcn_body(a_ref, x_ref, w_ref, b_ref, gam_ref, bet_ref, o_ref):
    A = a_ref[...]
    xg = x_ref[...]
    w = w_ref[...]
    h = lax.dot_general(xg, w, (((2,), (0,)), ((), ())),
                        preferred_element_type=jnp.float32)
    deg = jnp.sum(A, axis=2) + 1.0
    dinv = lax.rsqrt(deg)
    hs = h * dinv[:, :, None]
    out = lax.dot_general(A, hs, (((2,), (1,)), ((0,), (0,))),
                          preferred_element_type=jnp.float32) + hs
    out = out * dinv[:, :, None] + b_ref[...]
    mu = jnp.mean(out, axis=1, keepdims=True)
    var = jnp.mean((out - mu) * (out - mu), axis=1, keepdims=True)
    out = (out - mu) * lax.rsqrt(var + 1e-5) * gam_ref[...] + bet_ref[...]
    o_ref[...] = jnp.tanh(out).astype(jnp.bfloat16)


def _gcn_dense(A3, x, gcn_W, gcn_b, bn_gamma, bn_beta):
    return pl.pallas_call(
        _gcn_body,
        grid=(B // GB,),
        in_specs=[
            pl.BlockSpec((GB, N, N), lambda b: (b, 0, 0)),
            pl.BlockSpec((GB, N, D), lambda b: (b, 0, 0)),
            pl.BlockSpec((D, D), lambda b: (0, 0)),
            pl.BlockSpec((1, 1, D), lambda b: (0, 0, 0)),
            pl.BlockSpec((1, 1, D), lambda b: (0, 0, 0)),
            pl.BlockSpec((1, 1, D), lambda b: (0, 0, 0)),
        ],
        out_specs=pl.BlockSpec((GB, N, D), lambda b: (b, 0, 0)),
        out_shape=jax.ShapeDtypeStruct((B, N, D), jnp.bfloat16),
        compiler_params=pltpu.CompilerParams(
            dimension_semantics=("parallel",),
        ),
    )(A3, x, gcn_W, gcn_b.reshape(1, 1, D), bn_gamma.reshape(1, 1, D),
      bn_beta.reshape(1, 1, D))


def _enc_body(xs_ref, w1_ref, b1_ref, wmu_ref, bmu_ref, wlv_ref, blv_ref,
              dw1_ref, db1_ref, eps_ref, mean_ref, lv_ref, hd_ref, acc_ref):
    k = pl.program_id(0)

    @pl.when(k == 0)
    def _():
        acc_ref[...] = jnp.zeros_like(acc_ref)

    acc_ref[...] += jnp.dot(xs_ref[...], w1_ref[...],
                            preferred_element_type=jnp.float32)

    @pl.when(k == pl.num_programs(0) - 1)
    def _():
        h1 = jnp.maximum(acc_ref[...] + b1_ref[...], 0.0)
        mean = jnp.dot(h1, wmu_ref[...],
                       preferred_element_type=jnp.float32) + bmu_ref[...]
        lv = jnp.dot(h1, wlv_ref[...],
                     preferred_element_type=jnp.float32) + blv_ref[...]
        z = mean + jnp.exp(0.5 * lv) * eps_ref[...]
        hd = jnp.dot(z, dw1_ref[...],
                     preferred_element_type=jnp.float32) + db1_ref[...]
        mean_ref[...] = mean
        lv_ref[...] = lv
        hd_ref[...] = jnp.maximum(hd, 0.0).astype(jnp.bfloat16)


def _encoder(xs, enc_W1, enc_b1, enc_Wmu, enc_bmu, enc_Wlv, enc_blv,
             dec_W1, dec_b1, eps):
    nsteps = INPUT_DIM // KC
    return pl.pallas_call(
        _enc_body,
        grid=(nsteps,),
        in_specs=[
            pl.BlockSpec((B, KC), lambda k: (0, k)),
            pl.BlockSpec((KC, HIDDEN), lambda k: (k, 0)),
            pl.BlockSpec((1, HIDDEN), lambda k: (0, 0)),
            pl.BlockSpec((HIDDEN, L), lambda k: (0, 0)),
            pl.BlockSpec((1, L), lambda k: (0, 0)),
            pl.BlockSpec((HIDDEN, L), lambda k: (0, 0)),
            pl.BlockSpec((1, L), lambda k: (0, 0)),
            pl.BlockSpec((L, HIDDEN), lambda k: (0, 0)),
            pl.BlockSpec((1, HIDDEN), lambda k: (0, 0)),
            pl.BlockSpec((B, L), lambda k: (0, 0)),
        ],
        out_specs=[
            pl.BlockSpec((B, L), lambda k: (0, 0)),
            pl.BlockSpec((B, L), lambda k: (0, 0)),
            pl.BlockSpec((B, HIDDEN), lambda k: (0, 0)),
        ],
        out_shape=[
            jax.ShapeDtypeStruct((B, L), jnp.float32),
            jax.ShapeDtypeStruct((B, L), jnp.float32),
            jax.ShapeDtypeStruct((B, HIDDEN), jnp.bfloat16),
        ],
        scratch_shapes=[pltpu.VMEM((B, HIDDEN), jnp.float32)],
        compiler_params=pltpu.CompilerParams(
            dimension_semantics=("arbitrary",),
        ),
    )(xs, enc_W1, enc_b1.reshape(1, HIDDEN), enc_Wmu,
      enc_bmu.reshape(1, L), enc_Wlv, enc_blv.reshape(1, L),
      dec_W1, dec_b1.reshape(1, HIDDEN), eps)


def _dec_body(hd_ref, w2_ref, b2_ref, o_ref):
    o_ref[...] = jax.nn.sigmoid(
        jnp.dot(hd_ref[...], w2_ref[...],
                preferred_element_type=jnp.float32) + b2_ref[...])


def _decoder(hd, dec_W2, dec_b2):
    nsteps = INPUT_DIM // NC
    return pl.pallas_call(
        _dec_body,
        grid=(nsteps,),
        in_specs=[
            pl.BlockSpec((B, HIDDEN), lambda n: (0, 0)),
            pl.BlockSpec((HIDDEN, NC), lambda n: (0, n)),
            pl.BlockSpec((1, NC), lambda n: (0, n)),
        ],
        out_specs=pl.BlockSpec((B, NC), lambda n: (0, n)),
        out_shape=jax.ShapeDtypeStruct((B, INPUT_DIM), jnp.float32),
        compiler_params=pltpu.CompilerParams(
            dimension_semantics=("parallel",),
        ),
    )(hd, dec_W2, dec_b2.reshape(1, INPUT_DIM))


def kernel(x, edge_index, gcn_W, gcn_b, bn_gamma, bn_beta,
           enc_W1, enc_b1, enc_Wmu, enc_bmu, enc_Wlv, enc_blv,
           dec_W1, dec_b1, dec_W2, dec_b2, eps):
    A3 = _build_adj_fn()(edge_index)
    xs = _gcn_dense(A3, x, gcn_W, gcn_b, bn_gamma, bn_beta)
    xs = xs.reshape(B, INPUT_DIM)
    mean, log_var, hd = _encoder(xs, enc_W1, enc_b1,
                                 enc_Wmu, enc_bmu, enc_Wlv, enc_blv,
                                 dec_W1, dec_b1, eps)
    x_hat = _decoder(hd, dec_W2, dec_b2)
    return (x_hat, mean, log_var)


# speedup vs baseline: 137.6137x; 1.0873x over previous
import functools

import jax
import jax.numpy as jnp
from jax import lax
from jax.experimental import pallas as pl
from jax.experimental.pallas import tpu as pltpu
from jax.experimental.pallas import tpu_sc as plsc

B, N, D = 32, 256, 128
E = 8192
INPUT_DIM = N * D
HIDDEN = 512
L = 128

KC = 4096
NC = 4096


def _adj_body(edges_hbm, out_hbm, ei_v, a_v, sem):
    g = lax.axis_index("s") * 2 + lax.axis_index("c")
    cp = pltpu.async_copy(edges_hbm.at[g], ei_v, sem)
    zero = jnp.zeros((16,), jnp.float32)
    ones = jnp.full((16,), 1.0, jnp.float32)

    def zbody(r, carry):
        for j in range(N // 16):
            a_v[r, pl.ds(j * 16, 16)] = zero
        return carry

    lax.fori_loop(0, N, zbody, 0)
    cp.wait()

    def body(e, carry):
        base = e * 64
        for j in range(4):
            src = ei_v[0, pl.ds(base + j * 16, 16)]
            dst = ei_v[1, pl.ds(base + j * 16, 16)]
            plsc.addupdate_scatter(a_v, [dst, src], ones)
        return carry

    lax.fori_loop(0, E // 64, body, 0)
    pltpu.sync_copy(a_v, out_hbm.at[g])


@functools.cache
def _build_adj_fn():
    return functools.partial(
        pl.kernel,
        out_type=jax.ShapeDtypeStruct((B, N, N), jnp.float32),
        mesh=plsc.VectorSubcoreMesh(core_axis_name="c", subcore_axis_name="s"),
        scratch_types=[
            pltpu.VMEM((2, E), jnp.int32),
            pltpu.VMEM((N, N), jnp.float32),
            pltpu.SemaphoreType.DMA,
        ],
        compiler_params=pltpu.CompilerParams(needs_layout_passes=False),
    )(_adj_body)


GB = 8


def _gcn_body(a_ref, x_ref, w_ref, b_ref, gam_ref, bet_ref, o_ref):
    A = a_ref[...]
    xg = x_ref[...]
    w = w_ref[...]
    h = lax.dot_general(xg, w, (((2,), (0,)), ((), ())),
                        preferred_element_type=jnp.float32)
    deg = jnp.sum(A, axis=2) + 1.0
    dinv = lax.rsqrt(deg)
    hs = h * dinv[:, :, None]
    out = lax.dot_general(A, hs, (((2,), (1,)), ((0,), (0,))),
                          preferred_element_type=jnp.float32) + hs
    out = out * dinv[:, :, None] + b_ref[...]
    mu = jnp.mean(out, axis=1, keepdims=True)
    var = jnp.mean((out - mu) * (out - mu), axis=1, keepdims=True)
    out = (out - mu) * lax.rsqrt(var + 1e-5) * gam_ref[...] + bet_ref[...]
    o_ref[...] = jnp.tanh(out).astype(jnp.bfloat16).reshape(GB, N * D)


def _gcn_dense(A3, x, gcn_W, gcn_b, bn_gamma, bn_beta):
    return pl.pallas_call(
        _gcn_body,
        grid=(B // GB,),
        in_specs=[
            pl.BlockSpec((GB, N, N), lambda b: (b, 0, 0)),
            pl.BlockSpec((GB, N, D), lambda b: (b, 0, 0)),
            pl.BlockSpec((D, D), lambda b: (0, 0)),
            pl.BlockSpec((1, 1, D), lambda b: (0, 0, 0)),
            pl.BlockSpec((1, 1, D), lambda b: (0, 0, 0)),
            pl.BlockSpec((1, 1, D), lambda b: (0, 0, 0)),
        ],
        out_specs=pl.BlockSpec((GB, N * D), lambda b: (b, 0)),
        out_shape=jax.ShapeDtypeStruct((B, N * D), jnp.bfloat16),
        compiler_params=pltpu.CompilerParams(
            dimension_semantics=("parallel",),
        ),
    )(A3, x, gcn_W, gcn_b.reshape(1, 1, D), bn_gamma.reshape(1, 1, D),
      bn_beta.reshape(1, 1, D))


def _enc_body(xs_ref, w1_ref, b1_ref, wmu_ref, bmu_ref, wlv_ref, blv_ref,
              dw1_ref, db1_ref, eps_ref, mean_ref, lv_ref, hd_ref, acc_ref):
    k = pl.program_id(0)

    @pl.when(k == 0)
    def _():
        acc_ref[...] = jnp.zeros_like(acc_ref)

    acc_ref[...] += jnp.dot(xs_ref[...], w1_ref[...],
                            preferred_element_type=jnp.float32)

    @pl.when(k == pl.num_programs(0) - 1)
    def _():
        h1 = jnp.maximum(acc_ref[...] + b1_ref[...], 0.0)
        mean = jnp.dot(h1, wmu_ref[...],
                       preferred_element_type=jnp.float32) + bmu_ref[...]
        lv = jnp.dot(h1, wlv_ref[...],
                     preferred_element_type=jnp.float32) + blv_ref[...]
        z = mean + jnp.exp(0.5 * lv) * eps_ref[...]
        hd = jnp.dot(z, dw1_ref[...],
                     preferred_element_type=jnp.float32) + db1_ref[...]
        mean_ref[...] = mean
        lv_ref[...] = lv
        hd_ref[...] = jnp.maximum(hd, 0.0).astype(jnp.bfloat16)


def _encoder(xs, enc_W1, enc_b1, enc_Wmu, enc_bmu, enc_Wlv, enc_blv,
             dec_W1, dec_b1, eps):
    nsteps = INPUT_DIM // KC
    return pl.pallas_call(
        _enc_body,
        grid=(nsteps,),
        in_specs=[
            pl.BlockSpec((B, KC), lambda k: (0, k)),
            pl.BlockSpec((KC, HIDDEN), lambda k: (k, 0)),
            pl.BlockSpec((1, HIDDEN), lambda k: (0, 0)),
            pl.BlockSpec((HIDDEN, L), lambda k: (0, 0)),
            pl.BlockSpec((1, L), lambda k: (0, 0)),
            pl.BlockSpec((HIDDEN, L), lambda k: (0, 0)),
            pl.BlockSpec((1, L), lambda k: (0, 0)),
            pl.BlockSpec((L, HIDDEN), lambda k: (0, 0)),
            pl.BlockSpec((1, HIDDEN), lambda k: (0, 0)),
            pl.BlockSpec((B, L), lambda k: (0, 0)),
        ],
        out_specs=[
            pl.BlockSpec((B, L), lambda k: (0, 0)),
            pl.BlockSpec((B, L), lambda k: (0, 0)),
            pl.BlockSpec((B, HIDDEN), lambda k: (0, 0)),
        ],
        out_shape=[
            jax.ShapeDtypeStruct((B, L), jnp.float32),
            jax.ShapeDtypeStruct((B, L), jnp.float32),
            jax.ShapeDtypeStruct((B, HIDDEN), jnp.bfloat16),
        ],
        scratch_shapes=[pltpu.VMEM((B, HIDDEN), jnp.float32)],
        compiler_params=pltpu.CompilerParams(
            dimension_semantics=("arbitrary",),
        ),
    )(xs, enc_W1, enc_b1.reshape(1, HIDDEN), enc_Wmu,
      enc_bmu.reshape(1, L), enc_Wlv, enc_blv.reshape(1, L),
      dec_W1, dec_b1.reshape(1, HIDDEN), eps)


def _dec_body(hd_ref, w2_ref, b2_ref, o_ref):
    o_ref[...] = jax.nn.sigmoid(
        jnp.dot(hd_ref[...], w2_ref[...],
                preferred_element_type=jnp.float32) + b2_ref[...])


def _decoder(hd, dec_W2, dec_b2):
    nsteps = INPUT_DIM // NC
    return pl.pallas_call(
        _dec_body,
        grid=(nsteps,),
        in_specs=[
            pl.BlockSpec((B, HIDDEN), lambda n: (0, 0)),
            pl.BlockSpec((HIDDEN, NC), lambda n: (0, n)),
            pl.BlockSpec((1, NC), lambda n: (0, n)),
        ],
        out_specs=pl.BlockSpec((B, NC), lambda n: (0, n)),
        out_shape=jax.ShapeDtypeStruct((B, INPUT_DIM), jnp.float32),
        compiler_params=pltpu.CompilerParams(
            dimension_semantics=("parallel",),
        ),
    )(hd, dec_W2, dec_b2.reshape(1, INPUT_DIM))


def kernel(x, edge_index, gcn_W, gcn_b, bn_gamma, bn_beta,
           enc_W1, enc_b1, enc_Wmu, enc_bmu, enc_Wlv, enc_blv,
           dec_W1, dec_b1, dec_W2, dec_b2, eps):
    A3 = _build_adj_fn()(edge_index)
    xs = _gcn_dense(A3, x, gcn_W, gcn_b, bn_gamma, bn_beta)
    mean, log_var, hd = _encoder(xs, enc_W1, enc_b1,
                                 enc_Wmu, enc_bmu, enc_Wlv, enc_blv,
                                 dec_W1, dec_b1, eps)
    x_hat = _decoder(hd, dec_W2, dec_b2)
    return (x_hat, mean, log_var)


# speedup vs baseline: 137.7149x; 1.0007x over previous
import functools

import jax
import jax.numpy as jnp
from jax import lax
from jax.experimental import pallas as pl
from jax.experimental.pallas import tpu as pltpu
from jax.experimental.pallas import tpu_sc as plsc

B, N, D = 32, 256, 128
E = 8192
INPUT_DIM = N * D
HIDDEN = 512
L = 128

KC = 4096
NC = 4096


def _adj_body(edges_hbm, out_hbm, ei_v, a_v, sem):
    g = lax.axis_index("s") * 2 + lax.axis_index("c")
    cp = pltpu.async_copy(edges_hbm.at[g], ei_v, sem)
    zero = jnp.zeros((16,), jnp.float32)
    ones = jnp.full((16,), 1.0, jnp.float32)

    def zbody(r, carry):
        for j in range(N // 16):
            a_v[r, pl.ds(j * 16, 16)] = zero
        return carry

    lax.fori_loop(0, N, zbody, 0)
    cp.wait()

    def body(e, carry):
        base = e * 64
        for j in range(4):
            src = ei_v[0, pl.ds(base + j * 16, 16)]
            dst = ei_v[1, pl.ds(base + j * 16, 16)]
            plsc.addupdate_scatter(a_v, [dst, src], ones)
        return carry

    lax.fori_loop(0, E // 64, body, 0)
    pltpu.sync_copy(a_v, out_hbm.at[g])


@functools.cache
def _build_adj_fn():
    return functools.partial(
        pl.kernel,
        out_type=jax.ShapeDtypeStruct((B, N, N), jnp.float32),
        mesh=plsc.VectorSubcoreMesh(core_axis_name="c", subcore_axis_name="s"),
        scratch_types=[
            pltpu.VMEM((2, E), jnp.int32),
            pltpu.VMEM((N, N), jnp.float32),
            pltpu.SemaphoreType.DMA,
        ],
        compiler_params=pltpu.CompilerParams(needs_layout_passes=False),
    )(_adj_body)


GB = 16


def _gcn_body(a_ref, x_ref, w_ref, b_ref, gam_ref, bet_ref, o_ref):
    A = a_ref[...]
    xg = x_ref[...]
    w = w_ref[...]
    h = lax.dot_general(xg, w, (((2,), (0,)), ((), ())),
                        preferred_element_type=jnp.float32)
    deg = jnp.sum(A, axis=2) + 1.0
    dinv = lax.rsqrt(deg)
    hs = h * dinv[:, :, None]
    out = lax.dot_general(A, hs, (((2,), (1,)), ((0,), (0,))),
                          preferred_element_type=jnp.float32) + hs
    out = out * dinv[:, :, None] + b_ref[...]
    mu = jnp.mean(out, axis=1, keepdims=True)
    var = jnp.mean((out - mu) * (out - mu), axis=1, keepdims=True)
    out = (out - mu) * lax.rsqrt(var + 1e-5) * gam_ref[...] + bet_ref[...]
    o_ref[...] = jnp.tanh(out).astype(jnp.bfloat16).reshape(GB, N * D)


def _gcn_dense(A3, x, gcn_W, gcn_b, bn_gamma, bn_beta):
    return pl.pallas_call(
        _gcn_body,
        grid=(B // GB,),
        in_specs=[
            pl.BlockSpec((GB, N, N), lambda b: (b, 0, 0)),
            pl.BlockSpec((GB, N, D), lambda b: (b, 0, 0)),
            pl.BlockSpec((D, D), lambda b: (0, 0)),
            pl.BlockSpec((1, 1, D), lambda b: (0, 0, 0)),
            pl.BlockSpec((1, 1, D), lambda b: (0, 0, 0)),
            pl.BlockSpec((1, 1, D), lambda b: (0, 0, 0)),
        ],
        out_specs=pl.BlockSpec((GB, N * D), lambda b: (b, 0)),
        out_shape=jax.ShapeDtypeStruct((B, N * D), jnp.bfloat16),
        compiler_params=pltpu.CompilerParams(
            dimension_semantics=("parallel",),
        ),
    )(A3, x, gcn_W, gcn_b.reshape(1, 1, D), bn_gamma.reshape(1, 1, D),
      bn_beta.reshape(1, 1, D))


def _enc_body(xs_ref, w1_ref, b1_ref, wmu_ref, bmu_ref, wlv_ref, blv_ref,
              dw1_ref, db1_ref, eps_ref, mean_ref, lv_ref, hd_ref, acc_ref):
    k = pl.program_id(0)

    @pl.when(k == 0)
    def _():
        acc_ref[...] = jnp.zeros_like(acc_ref)

    acc_ref[...] += jnp.dot(xs_ref[...], w1_ref[...],
                            preferred_element_type=jnp.float32)

    @pl.when(k == pl.num_programs(0) - 1)
    def _():
        h1 = jnp.maximum(acc_ref[...] + b1_ref[...], 0.0)
        mean = jnp.dot(h1, wmu_ref[...],
                       preferred_element_type=jnp.float32) + bmu_ref[...]
        lv = jnp.dot(h1, wlv_ref[...],
                     preferred_element_type=jnp.float32) + blv_ref[...]
        z = mean + jnp.exp(0.5 * lv) * eps_ref[...]
        hd = jnp.dot(z, dw1_ref[...],
                     preferred_element_type=jnp.float32) + db1_ref[...]
        mean_ref[...] = mean
        lv_ref[...] = lv
        hd_ref[...] = jnp.maximum(hd, 0.0).astype(jnp.bfloat16)


def _encoder(xs, enc_W1, enc_b1, enc_Wmu, enc_bmu, enc_Wlv, enc_blv,
             dec_W1, dec_b1, eps):
    nsteps = INPUT_DIM // KC
    return pl.pallas_call(
        _enc_body,
        grid=(nsteps,),
        in_specs=[
            pl.BlockSpec((B, KC), lambda k: (0, k)),
            pl.BlockSpec((KC, HIDDEN), lambda k: (k, 0)),
            pl.BlockSpec((1, HIDDEN), lambda k: (0, 0)),
            pl.BlockSpec((HIDDEN, L), lambda k: (0, 0)),
            pl.BlockSpec((1, L), lambda k: (0, 0)),
            pl.BlockSpec((HIDDEN, L), lambda k: (0, 0)),
            pl.BlockSpec((1, L), lambda k: (0, 0)),
            pl.BlockSpec((L, HIDDEN), lambda k: (0, 0)),
            pl.BlockSpec((1, HIDDEN), lambda k: (0, 0)),
            pl.BlockSpec((B, L), lambda k: (0, 0)),
        ],
        out_specs=[
            pl.BlockSpec((B, L), lambda k: (0, 0)),
            pl.BlockSpec((B, L), lambda k: (0, 0)),
            pl.BlockSpec((B, HIDDEN), lambda k: (0, 0)),
        ],
        out_shape=[
            jax.ShapeDtypeStruct((B, L), jnp.float32),
            jax.ShapeDtypeStruct((B, L), jnp.float32),
            jax.ShapeDtypeStruct((B, HIDDEN), jnp.bfloat16),
        ],
        scratch_shapes=[pltpu.VMEM((B, HIDDEN), jnp.float32)],
        compiler_params=pltpu.CompilerParams(
            dimension_semantics=("arbitrary",),
        ),
    )(xs, enc_W1, enc_b1.reshape(1, HIDDEN), enc_Wmu,
      enc_bmu.reshape(1, L), enc_Wlv, enc_blv.reshape(1, L),
      dec_W1, dec_b1.reshape(1, HIDDEN), eps)


def _dec_body(hd_ref, w2_ref, b2_ref, o_ref):
    o_ref[...] = jax.nn.sigmoid(
        jnp.dot(hd_ref[...], w2_ref[...],
                preferred_element_type=jnp.float32) + b2_ref[...])


def _decoder(hd, dec_W2, dec_b2):
    nsteps = INPUT_DIM // NC
    return pl.pallas_call(
        _dec_body,
        grid=(nsteps,),
        in_specs=[
            pl.BlockSpec((B, HIDDEN), lambda n: (0, 0)),
            pl.BlockSpec((HIDDEN, NC), lambda n: (0, n)),
            pl.BlockSpec((1, NC), lambda n: (0, n)),
        ],
        out_specs=pl.BlockSpec((B, NC), lambda n: (0, n)),
        out_shape=jax.ShapeDtypeStruct((B, INPUT_DIM), jnp.float32),
        compiler_params=pltpu.CompilerParams(
            dimension_semantics=("parallel",),
        ),
    )(hd, dec_W2, dec_b2.reshape(1, INPUT_DIM))


def kernel(x, edge_index, gcn_W, gcn_b, bn_gamma, bn_beta,
           enc_W1, enc_b1, enc_Wmu, enc_bmu, enc_Wlv, enc_blv,
           dec_W1, dec_b1, dec_W2, dec_b2, eps):
    A3 = _build_adj_fn()(edge_index)
    xs = _gcn_dense(A3, x, gcn_W, gcn_b, bn_gamma, bn_beta)
    mean, log_var, hd = _encoder(xs, enc_W1, enc_b1,
                                 enc_Wmu, enc_bmu, enc_Wlv, enc_blv,
                                 dec_W1, dec_b1, eps)
    x_hat = _decoder(hd, dec_W2, dec_b2)
    return (x_hat, mean, log_var)


# speedup vs baseline: 141.2156x; 1.0254x over previous
import functools

import jax
import jax.numpy as jnp
from jax import lax
from jax.experimental import pallas as pl
from jax.experimental.pallas import tpu as pltpu
from jax.experimental.pallas import tpu_sc as plsc

B, N, D = 32, 256, 128
E = 8192
INPUT_DIM = N * D
HIDDEN = 512
L = 128

GB = 16
KC = 4096
NC = 4096
GS = B // GB
ES = INPUT_DIM // KC
DS = INPUT_DIM // NC


def _adj_body(edges_hbm, out_hbm, ei_v, a_v, sem):
    g = lax.axis_index("s") * 2 + lax.axis_index("c")
    cp = pltpu.async_copy(edges_hbm.at[g], ei_v, sem)
    zero = jnp.zeros((16,), jnp.float32)
    ones = jnp.full((16,), 1.0, jnp.float32)

    def zbody(r, carry):
        for j in range(N // 16):
            a_v[r, pl.ds(j * 16, 16)] = zero
        return carry

    lax.fori_loop(0, N, zbody, 0)
    cp.wait()

    def body(e, carry):
        base = e * 64
        for j in range(4):
            src = ei_v[0, pl.ds(base + j * 16, 16)]
            dst = ei_v[1, pl.ds(base + j * 16, 16)]
            plsc.addupdate_scatter(a_v, [dst, src], ones)
        return carry

    lax.fori_loop(0, E // 64, body, 0)
    pltpu.sync_copy(a_v, out_hbm.at[g])


@functools.cache
def _build_adj_fn():
    return functools.partial(
        pl.kernel,
        out_type=jax.ShapeDtypeStruct((B, N, N), jnp.float32),
        mesh=plsc.VectorSubcoreMesh(core_axis_name="c", subcore_axis_name="s"),
        scratch_types=[
            pltpu.VMEM((2, E), jnp.int32),
            pltpu.VMEM((N, N), jnp.float32),
            pltpu.SemaphoreType.DMA,
        ],
        compiler_params=pltpu.CompilerParams(needs_layout_passes=False),
    )(_adj_body)


def _fused_body(a_ref, x_ref, w_ref, b_ref, gam_ref, bet_ref,
                w1_ref, b1_ref, wmu_ref, bmu_ref, wlv_ref, blv_ref,
                dw1_ref, db1_ref, eps_ref, w2_ref, b2_ref,
                mean_ref, lv_ref, xhat_ref,
                xs_s, acc_s, hd_s):
    s = pl.program_id(0)

    @pl.when(s < GS)
    def _gcn():
        A = a_ref[...]
        xg = x_ref[...]
        h = lax.dot_general(xg, w_ref[...], (((2,), (0,)), ((), ())),
                            preferred_element_type=jnp.float32)
        deg = jnp.sum(A, axis=2) + 1.0
        dinv = lax.rsqrt(deg)
        hs = h * dinv[:, :, None]
        out = lax.dot_general(A, hs, (((2,), (1,)), ((0,), (0,))),
                              preferred_element_type=jnp.float32) + hs
        out = out * dinv[:, :, None] + b_ref[...]
        mu = jnp.mean(out, axis=1, keepdims=True)
        var = jnp.mean((out - mu) * (out - mu), axis=1, keepdims=True)
        out = (out - mu) * lax.rsqrt(var + 1e-5) * gam_ref[...] + bet_ref[...]
        xs_s[pl.ds(s * GB, GB), :] = (
            jnp.tanh(out).astype(jnp.bfloat16).reshape(GB, N * D))

    @pl.when(s == GS)
    def _init():
        acc_s[...] = jnp.zeros_like(acc_s)

    @pl.when((s >= GS) & (s < GS + ES))
    def _enc():
        k = s - GS
        xs_blk = xs_s[:, pl.ds(k * KC, KC)]
        acc_s[...] += jnp.dot(xs_blk, w1_ref[...],
                              preferred_element_type=jnp.float32)

    @pl.when(s == GS + ES - 1)
    def _head():
        h1 = jnp.maximum(acc_s[...] + b1_ref[...], 0.0)
        mean = jnp.dot(h1, wmu_ref[...],
                       preferred_element_type=jnp.float32) + bmu_ref[...]
        lv = jnp.dot(h1, wlv_ref[...],
                     preferred_element_type=jnp.float32) + blv_ref[...]
        z = mean + jnp.exp(0.5 * lv) * eps_ref[...]
        hd = jnp.dot(z, dw1_ref[...],
                     preferred_element_type=jnp.float32) + db1_ref[...]
        mean_ref[...] = mean
        lv_ref[...] = lv
        hd_s[...] = jnp.maximum(hd, 0.0).astype(jnp.bfloat16)

    @pl.when(s >= GS + ES)
    def _dec():
        xhat_ref[...] = jax.nn.sigmoid(
            jnp.dot(hd_s[...], w2_ref[...],
                    preferred_element_type=jnp.float32) + b2_ref[...])


def _fused(A3, x, gcn_W, gcn_b, bn_gamma, bn_beta,
           enc_W1, enc_b1, enc_Wmu, enc_bmu, enc_Wlv, enc_blv,
           dec_W1, dec_b1, eps, dec_W2, dec_b2):
    nsteps = GS + ES + DS
    c0 = lambda s: (0, 0)
    c000 = lambda s: (0, 0, 0)
    return pl.pallas_call(
        _fused_body,
        grid=(nsteps,),
        in_specs=[
            pl.BlockSpec((GB, N, N), lambda s: (jnp.minimum(s, GS - 1), 0, 0)),
            pl.BlockSpec((GB, N, D), lambda s: (jnp.minimum(s, GS - 1), 0, 0)),
            pl.BlockSpec((D, D), c0),
            pl.BlockSpec((1, 1, D), c000),
            pl.BlockSpec((1, 1, D), c000),
            pl.BlockSpec((1, 1, D), c000),
            pl.BlockSpec((KC, HIDDEN),
                         lambda s: (jnp.clip(s - GS, 0, ES - 1), 0)),
            pl.BlockSpec((1, HIDDEN), c0),
            pl.BlockSpec((HIDDEN, L), c0),
            pl.BlockSpec((1, L), c0),
            pl.BlockSpec((HIDDEN, L), c0),
            pl.BlockSpec((1, L), c0),
            pl.BlockSpec((L, HIDDEN), c0),
            pl.BlockSpec((1, HIDDEN), c0),
            pl.BlockSpec((B, L), c0),
            pl.BlockSpec((HIDDEN, NC),
                         lambda s: (0, jnp.clip(s - GS - ES, 0, DS - 1))),
            pl.BlockSpec((1, NC),
                         lambda s: (0, jnp.clip(s - GS - ES, 0, DS - 1))),
        ],
        out_specs=[
            pl.BlockSpec((B, L), c0),
            pl.BlockSpec((B, L), c0),
            pl.BlockSpec((B, NC),
                         lambda s: (0, jnp.clip(s - GS - ES, 0, DS - 1))),
        ],
        out_shape=[
            jax.ShapeDtypeStruct((B, L), jnp.float32),
            jax.ShapeDtypeStruct((B, L), jnp.float32),
            jax.ShapeDtypeStruct((B, INPUT_DIM), jnp.float32),
        ],
        scratch_shapes=[
            pltpu.VMEM((B, INPUT_DIM), jnp.bfloat16),
            pltpu.VMEM((B, HIDDEN), jnp.float32),
            pltpu.VMEM((B, HIDDEN), jnp.bfloat16),
        ],
        compiler_params=pltpu.CompilerParams(
            dimension_semantics=("arbitrary",),
        ),
    )(A3, x, gcn_W, gcn_b.reshape(1, 1, D), bn_gamma.reshape(1, 1, D),
      bn_beta.reshape(1, 1, D), enc_W1, enc_b1.reshape(1, HIDDEN),
      enc_Wmu, enc_bmu.reshape(1, L), enc_Wlv, enc_blv.reshape(1, L),
      dec_W1, dec_b1.reshape(1, HIDDEN), eps, dec_W2,
      dec_b2.reshape(1, INPUT_DIM))


def kernel(x, edge_index, gcn_W, gcn_b, bn_gamma, bn_beta,
           enc_W1, enc_b1, enc_Wmu, enc_bmu, enc_Wlv, enc_blv,
           dec_W1, dec_b1, dec_W2, dec_b2, eps):
    A3 = _build_adj_fn()(edge_index)
    mean, log_var, x_hat = _fused(
        A3, x, gcn_W, gcn_b, bn_gamma, bn_beta,
        enc_W1, enc_b1, enc_Wmu, enc_bmu, enc_Wlv, enc_blv,
        dec_W1, dec_b1, eps, dec_W2, dec_b2)
    return (x_hat, mean, log_var)


# speedup vs baseline: 142.1013x; 1.0063x over previous
import functools

import jax
import jax.numpy as jnp
from jax import lax
from jax.experimental import pallas as pl
from jax.experimental.pallas import tpu as pltpu
from jax.experimental.pallas import tpu_sc as plsc

B, N, D = 32, 256, 128
E = 8192
INPUT_DIM = N * D
HIDDEN = 512
L = 128

GB = 16
KC = 4096
NC = 4096
GS = B // GB
ES = INPUT_DIM // KC
DS = INPUT_DIM // NC


def _adj_body(edges_hbm, out_hbm, ei_v, a_v, sem):
    g = lax.axis_index("s") * 2 + lax.axis_index("c")
    cp = pltpu.async_copy(edges_hbm.at[g], ei_v, sem)
    zero = jnp.zeros((16,), jnp.int32)
    one = jnp.full((16,), 1, jnp.int32)
    hi_one = jnp.full((16,), 65536, jnp.int32)

    def zbody(r, carry):
        for j in range(N // 32):
            a_v[r, pl.ds(j * 16, 16)] = zero
        return carry

    lax.fori_loop(0, N, zbody, 0)
    cp.wait()

    def body(e, carry):
        base = e * 64
        for j in range(4):
            src = ei_v[0, pl.ds(base + j * 16, 16)]
            dst = ei_v[1, pl.ds(base + j * 16, 16)]
            val = jnp.where((src & 1) == 1, hi_one, one)
            plsc.addupdate_scatter(a_v, [dst, src >> 1], val)
        return carry

    lax.fori_loop(0, E // 64, body, 0)
    pltpu.sync_copy(a_v, out_hbm.at[g])


@functools.cache
def _build_adj_fn():
    return functools.partial(
        pl.kernel,
        out_type=jax.ShapeDtypeStruct((B, N, N // 2), jnp.int32),
        mesh=plsc.VectorSubcoreMesh(core_axis_name="c", subcore_axis_name="s"),
        scratch_types=[
            pltpu.VMEM((2, E), jnp.int32),
            pltpu.VMEM((N, N // 2), jnp.int32),
            pltpu.SemaphoreType.DMA,
        ],
        compiler_params=pltpu.CompilerParams(needs_layout_passes=False),
    )(_adj_body)


def _fused_body(a_ref, x_ref, w_ref, b_ref, gam_ref, bet_ref,
                w1_ref, b1_ref, wmu_ref, bmu_ref, wlv_ref, blv_ref,
                dw1_ref, db1_ref, eps_ref, w2_ref, b2_ref,
                mean_ref, lv_ref, xhat_ref,
                xs_s, acc_s, hd_s):
    s = pl.program_id(0)

    @pl.when(s < GS)
    def _gcn():
        a16 = a_ref[...]
        lo = (a16 & 0xFFFF).astype(jnp.float32)
        hi = ((a16 >> 16) & 0xFFFF).astype(jnp.float32)
        xg = x_ref[...]
        h = lax.dot_general(xg, w_ref[...], (((2,), (0,)), ((), ())),
                            preferred_element_type=jnp.float32)
        deg = jnp.sum(lo, axis=2) + jnp.sum(hi, axis=2) + 1.0
        dinv = lax.rsqrt(deg)
        hs = h * dinv[:, :, None]
        hs2 = hs.reshape(GB, N // 2, 2, D)
        hs_even = hs2[:, :, 0, :]
        hs_odd = hs2[:, :, 1, :]
        bd = (((2,), (1,)), ((0,), (0,)))
        out = (lax.dot_general(lo, hs_even, bd,
                               preferred_element_type=jnp.float32)
               + lax.dot_general(hi, hs_odd, bd,
                                 preferred_element_type=jnp.float32)
               + hs)
        out = out * dinv[:, :, None] + b_ref[...]
        mu = jnp.mean(out, axis=1, keepdims=True)
        var = jnp.mean((out - mu) * (out - mu), axis=1, keepdims=True)
        out = (out - mu) * lax.rsqrt(var + 1e-5) * gam_ref[...] + bet_ref[...]
        xs_s[pl.ds(s * GB, GB), :] = (
            jnp.tanh(out).astype(jnp.bfloat16).reshape(GB, N * D))

    @pl.when(s == GS)
    def _init():
        acc_s[...] = jnp.zeros_like(acc_s)

    @pl.when((s >= GS) & (s < GS + ES))
    def _enc():
        k = s - GS
        xs_blk = xs_s[:, pl.ds(k * KC, KC)]
        acc_s[...] += jnp.dot(xs_blk, w1_ref[...],
                              preferred_element_type=jnp.float32)

    @pl.when(s == GS + ES - 1)
    def _head():
        h1 = jnp.maximum(acc_s[...] + b1_ref[...], 0.0)
        mean = jnp.dot(h1, wmu_ref[...],
                       preferred_element_type=jnp.float32) + bmu_ref[...]
        lv = jnp.dot(h1, wlv_ref[...],
                     preferred_element_type=jnp.float32) + blv_ref[...]
        z = mean + jnp.exp(0.5 * lv) * eps_ref[...]
        hd = jnp.dot(z, dw1_ref[...],
                     preferred_element_type=jnp.float32) + db1_ref[...]
        mean_ref[...] = mean
        lv_ref[...] = lv
        hd_s[...] = jnp.maximum(hd, 0.0).astype(jnp.bfloat16)

    @pl.when(s >= GS + ES)
    def _dec():
        xhat_ref[...] = jax.nn.sigmoid(
            jnp.dot(hd_s[...], w2_ref[...],
                    preferred_element_type=jnp.float32) + b2_ref[...])


def _fused(A3, x, gcn_W, gcn_b, bn_gamma, bn_beta,
           enc_W1, enc_b1, enc_Wmu, enc_bmu, enc_Wlv, enc_blv,
           dec_W1, dec_b1, eps, dec_W2, dec_b2):
    nsteps = GS + ES + DS
    c0 = lambda s: (0, 0)
    c000 = lambda s: (0, 0, 0)
    return pl.pallas_call(
        _fused_body,
        grid=(nsteps,),
        in_specs=[
            pl.BlockSpec((GB, N, N // 2),
                         lambda s: (jnp.minimum(s, GS - 1), 0, 0)),
            pl.BlockSpec((GB, N, D), lambda s: (jnp.minimum(s, GS - 1), 0, 0)),
            pl.BlockSpec((D, D), c0),
            pl.BlockSpec((1, 1, D), c000),
            pl.BlockSpec((1, 1, D), c000),
            pl.BlockSpec((1, 1, D), c000),
            pl.BlockSpec((KC, HIDDEN),
                         lambda s: (jnp.clip(s - GS, 0, ES - 1), 0)),
            pl.BlockSpec((1, HIDDEN), c0),
            pl.BlockSpec((HIDDEN, L), c0),
            pl.BlockSpec((1, L), c0),
            pl.BlockSpec((HIDDEN, L), c0),
            pl.BlockSpec((1, L), c0),
            pl.BlockSpec((L, HIDDEN), c0),
            pl.BlockSpec((1, HIDDEN), c0),
            pl.BlockSpec((B, L), c0),
            pl.BlockSpec((HIDDEN, NC),
                         lambda s: (0, jnp.clip(s - GS - ES, 0, DS - 1))),
            pl.BlockSpec((1, NC),
                         lambda s: (0, jnp.clip(s - GS - ES, 0, DS - 1))),
        ],
        out_specs=[
            pl.BlockSpec((B, L), c0),
            pl.BlockSpec((B, L), c0),
            pl.BlockSpec((B, NC),
                         lambda s: (0, jnp.clip(s - GS - ES, 0, DS - 1))),
        ],
        out_shape=[
            jax.ShapeDtypeStruct((B, L), jnp.float32),
            jax.ShapeDtypeStruct((B, L), jnp.float32),
            jax.ShapeDtypeStruct((B, INPUT_DIM), jnp.float32),
        ],
        scratch_shapes=[
            pltpu.VMEM((B, INPUT_DIM), jnp.bfloat16),
            pltpu.VMEM((B, HIDDEN), jnp.float32),
            pltpu.VMEM((B, HIDDEN), jnp.bfloat16),
        ],
        compiler_params=pltpu.CompilerParams(
            dimension_semantics=("arbitrary",),
        ),
    )(A3, x, gcn_W, gcn_b.reshape(1, 1, D), bn_gamma.reshape(1, 1, D),
      bn_beta.reshape(1, 1, D), enc_W1, enc_b1.reshape(1, HIDDEN),
      enc_Wmu, enc_bmu.reshape(1, L), enc_Wlv, enc_blv.reshape(1, L),
      dec_W1, dec_b1.reshape(1, HIDDEN), eps, dec_W2,
      dec_b2.reshape(1, INPUT_DIM))


def kernel(x, edge_index, gcn_W, gcn_b, bn_gamma, bn_beta,
           enc_W1, enc_b1, enc_Wmu, enc_bmu, enc_Wlv, enc_blv,
           dec_W1, dec_b1, dec_W2, dec_b2, eps):
    A3 = _build_adj_fn()(edge_index)
    mean, log_var, x_hat = _fused(
        A3, x, gcn_W, gcn_b, bn_gamma, bn_beta,
        enc_W1, enc_b1, enc_Wmu, enc_bmu, enc_Wlv, enc_blv,
        dec_W1, dec_b1, eps, dec_W2, dec_b2)
    return (x_hat, mean, log_var)


# speedup vs baseline: 143.9666x; 1.0131x over previous
import functools

import jax
import jax.numpy as jnp
from jax import lax
from jax.experimental import pallas as pl
from jax.experimental.pallas import tpu as pltpu
from jax.experimental.pallas import tpu_sc as plsc

B, N, D = 32, 256, 128
E = 8192
INPUT_DIM = N * D
HIDDEN = 512
L = 128

GB = 8
KC = 4096
NC = 4096
GS = B // GB
ES = INPUT_DIM // KC
DS = INPUT_DIM // NC


def _adj_body(edges_hbm, out_hbm, ei_v, a_v, sem):
    g = lax.axis_index("s") * 2 + lax.axis_index("c")
    cp = pltpu.async_copy(edges_hbm.at[g], ei_v, sem)
    zero = jnp.zeros((16,), jnp.int32)
    one = jnp.full((16,), 1, jnp.int32)
    hi_one = jnp.full((16,), 65536, jnp.int32)

    def zbody(r, carry):
        for j in range(N // 32):
            a_v[r, pl.ds(j * 16, 16)] = zero
        return carry

    lax.fori_loop(0, N, zbody, 0)
    cp.wait()

    def body(e, carry):
        base = e * 64
        for j in range(4):
            src = ei_v[0, pl.ds(base + j * 16, 16)]
            dst = ei_v[1, pl.ds(base + j * 16, 16)]
            val = jnp.where((src & 1) == 1, hi_one, one)
            plsc.addupdate_scatter(a_v, [dst, src >> 1], val)
        return carry

    lax.fori_loop(0, E // 64, body, 0)
    pltpu.sync_copy(a_v, out_hbm.at[g])


@functools.cache
def _build_adj_fn():
    return functools.partial(
        pl.kernel,
        out_type=jax.ShapeDtypeStruct((B, N, N // 2), jnp.int32),
        mesh=plsc.VectorSubcoreMesh(core_axis_name="c", subcore_axis_name="s"),
        scratch_types=[
            pltpu.VMEM((2, E), jnp.int32),
            pltpu.VMEM((N, N // 2), jnp.int32),
            pltpu.SemaphoreType.DMA,
        ],
        compiler_params=pltpu.CompilerParams(needs_layout_passes=False),
    )(_adj_body)


def _fused_body(a_ref, x_ref, w_ref, b_ref, gam_ref, bet_ref,
                w1_ref, b1_ref, wmu_ref, bmu_ref, wlv_ref, blv_ref,
                dw1_ref, db1_ref, eps_ref, w2_ref, b2_ref,
                mean_ref, lv_ref, xhat_ref,
                xs_s, acc_s, hd_s):
    s = pl.program_id(0)

    @pl.when(s < GS)
    def _gcn():
        a16 = a_ref[...]
        lo = (a16 & 0xFFFF).astype(jnp.float32)
        hi = ((a16 >> 16) & 0xFFFF).astype(jnp.float32)
        xg = x_ref[...]
        h = lax.dot_general(xg, w_ref[...], (((2,), (0,)), ((), ())),
                            preferred_element_type=jnp.float32)
        deg = jnp.sum(lo, axis=2) + jnp.sum(hi, axis=2) + 1.0
        dinv = lax.rsqrt(deg)
        hs = h * dinv[:, :, None]
        hs2 = hs.reshape(GB, N // 2, 2, D)
        hs_even = hs2[:, :, 0, :]
        hs_odd = hs2[:, :, 1, :]
        bd = (((2,), (1,)), ((0,), (0,)))
        out = (lax.dot_general(lo, hs_even, bd,
                               preferred_element_type=jnp.float32)
               + lax.dot_general(hi, hs_odd, bd,
                                 preferred_element_type=jnp.float32)
               + hs)
        out = out * dinv[:, :, None] + b_ref[...]
        mu = jnp.mean(out, axis=1, keepdims=True)
        var = jnp.mean((out - mu) * (out - mu), axis=1, keepdims=True)
        out = (out - mu) * lax.rsqrt(var + 1e-5) * gam_ref[...] + bet_ref[...]
        xs_s[pl.ds(s * GB, GB), :] = (
            jnp.tanh(out).astype(jnp.bfloat16).reshape(GB, N * D))

    @pl.when(s == GS)
    def _init():
        acc_s[...] = jnp.zeros_like(acc_s)

    @pl.when((s >= GS) & (s < GS + ES))
    def _enc():
        k = s - GS
        xs_blk = xs_s[:, pl.ds(k * KC, KC)]
        acc_s[...] += jnp.dot(xs_blk, w1_ref[...],
                              preferred_element_type=jnp.float32)

    @pl.when(s == GS + ES - 1)
    def _head():
        h1 = jnp.maximum(acc_s[...] + b1_ref[...], 0.0)
        mean = jnp.dot(h1, wmu_ref[...],
                       preferred_element_type=jnp.float32) + bmu_ref[...]
        lv = jnp.dot(h1, wlv_ref[...],
                     preferred_element_type=jnp.float32) + blv_ref[...]
        z = mean + jnp.exp(0.5 * lv) * eps_ref[...]
        hd = jnp.dot(z, dw1_ref[...],
                     preferred_element_type=jnp.float32) + db1_ref[...]
        mean_ref[...] = mean
        lv_ref[...] = lv
        hd_s[...] = jnp.maximum(hd, 0.0).astype(jnp.bfloat16)

    @pl.when(s >= GS + ES)
    def _dec():
        xhat_ref[...] = jax.nn.sigmoid(
            jnp.dot(hd_s[...], w2_ref[...],
                    preferred_element_type=jnp.float32) + b2_ref[...])


def _fused(A3, x, gcn_W, gcn_b, bn_gamma, bn_beta,
           enc_W1, enc_b1, enc_Wmu, enc_bmu, enc_Wlv, enc_blv,
           dec_W1, dec_b1, eps, dec_W2, dec_b2):
    nsteps = GS + ES + DS
    c0 = lambda s: (0, 0)
    c000 = lambda s: (0, 0, 0)
    return pl.pallas_call(
        _fused_body,
        grid=(nsteps,),
        in_specs=[
            pl.BlockSpec((GB, N, N // 2),
                         lambda s: (jnp.minimum(s, GS - 1), 0, 0)),
            pl.BlockSpec((GB, N, D), lambda s: (jnp.minimum(s, GS - 1), 0, 0)),
            pl.BlockSpec((D, D), c0),
            pl.BlockSpec((1, 1, D), c000),
            pl.BlockSpec((1, 1, D), c000),
            pl.BlockSpec((1, 1, D), c000),
            pl.BlockSpec((KC, HIDDEN),
                         lambda s: (jnp.clip(s - GS, 0, ES - 1), 0)),
            pl.BlockSpec((1, HIDDEN), c0),
            pl.BlockSpec((HIDDEN, L), c0),
            pl.BlockSpec((1, L), c0),
            pl.BlockSpec((HIDDEN, L), c0),
            pl.BlockSpec((1, L), c0),
            pl.BlockSpec((L, HIDDEN), c0),
            pl.BlockSpec((1, HIDDEN), c0),
            pl.BlockSpec((B, L), c0),
            pl.BlockSpec((HIDDEN, NC),
                         lambda s: (0, jnp.clip(s - GS - ES, 0, DS - 1))),
            pl.BlockSpec((1, NC),
                         lambda s: (0, jnp.clip(s - GS - ES, 0, DS - 1))),
        ],
        out_specs=[
            pl.BlockSpec((B, L), c0),
            pl.BlockSpec((B, L), c0),
            pl.BlockSpec((B, NC),
                         lambda s: (0, jnp.clip(s - GS - ES, 0, DS - 1))),
        ],
        out_shape=[
            jax.ShapeDtypeStruct((B, L), jnp.float32),
            jax.ShapeDtypeStruct((B, L), jnp.float32),
            jax.ShapeDtypeStruct((B, INPUT_DIM), jnp.float32),
        ],
        scratch_shapes=[
            pltpu.VMEM((B, INPUT_DIM), jnp.bfloat16),
            pltpu.VMEM((B, HIDDEN), jnp.float32),
            pltpu.VMEM((B, HIDDEN), jnp.bfloat16),
        ],
        compiler_params=pltpu.CompilerParams(
            dimension_semantics=("arbitrary",),
        ),
    )(A3, x, gcn_W, gcn_b.reshape(1, 1, D), bn_gamma.reshape(1, 1, D),
      bn_beta.reshape(1, 1, D), enc_W1, enc_b1.reshape(1, HIDDEN),
      enc_Wmu, enc_bmu.reshape(1, L), enc_Wlv, enc_blv.reshape(1, L),
      dec_W1, dec_b1.reshape(1, HIDDEN), eps, dec_W2,
      dec_b2.reshape(1, INPUT_DIM))


def kernel(x, edge_index, gcn_W, gcn_b, bn_gamma, bn_beta,
           enc_W1, enc_b1, enc_Wmu, enc_bmu, enc_Wlv, enc_blv,
           dec_W1, dec_b1, dec_W2, dec_b2, eps):
    A3 = _build_adj_fn()(edge_index)
    mean, log_var, x_hat = _fused(
        A3, x, gcn_W, gcn_b, bn_gamma, bn_beta,
        enc_W1, enc_b1, enc_Wmu, enc_bmu, enc_Wlv, enc_blv,
        dec_W1, dec_b1, eps, dec_W2, dec_b2)
    return (x_hat, mean, log_var)


# speedup vs baseline: 144.4585x; 1.0034x over previous
import functools

import jax
import jax.numpy as jnp
from jax import lax
from jax.experimental import pallas as pl
from jax.experimental.pallas import tpu as pltpu
from jax.experimental.pallas import tpu_sc as plsc

B, N, D = 32, 256, 128
E = 8192
INPUT_DIM = N * D
HIDDEN = 512
L = 128

GB = 8
KC = 4096
NC = 4096
GS = B // GB
ES = INPUT_DIM // KC
DS = INPUT_DIM // NC


def _adj_body(edges_hbm, out_hbm, ei_v, a_v, sem):
    g = lax.axis_index("s") * 2 + lax.axis_index("c")
    cp = pltpu.async_copy(edges_hbm.at[g], ei_v, sem)
    zero = jnp.zeros((16,), jnp.int32)
    one = jnp.full((16,), 1, jnp.int32)
    hi_one = jnp.full((16,), 65536, jnp.int32)

    def zbody(r, carry):
        for j in range(N // 32):
            a_v[r, pl.ds(j * 16, 16)] = zero
        return carry

    lax.fori_loop(0, N, zbody, 0)
    cp.wait()

    def body(e, carry):
        base = e * 128
        for j in range(8):
            src = ei_v[0, pl.ds(base + j * 16, 16)]
            dst = ei_v[1, pl.ds(base + j * 16, 16)]
            val = jnp.where((src & 1) == 1, hi_one, one)
            plsc.addupdate_scatter(a_v, [dst, src >> 1], val)
        return carry

    lax.fori_loop(0, E // 128, body, 0)
    pltpu.sync_copy(a_v, out_hbm.at[g])


@functools.cache
def _build_adj_fn():
    return functools.partial(
        pl.kernel,
        out_type=jax.ShapeDtypeStruct((B, N, N // 2), jnp.int32),
        mesh=plsc.VectorSubcoreMesh(core_axis_name="c", subcore_axis_name="s"),
        scratch_types=[
            pltpu.VMEM((2, E), jnp.int32),
            pltpu.VMEM((N, N // 2), jnp.int32),
            pltpu.SemaphoreType.DMA,
        ],
        compiler_params=pltpu.CompilerParams(needs_layout_passes=False),
    )(_adj_body)


def _fused_body(a_ref, x_ref, w_ref, b_ref, gam_ref, bet_ref,
                w1_ref, b1_ref, wmu_ref, bmu_ref, wlv_ref, blv_ref,
                dw1_ref, db1_ref, eps_ref, w2_ref, b2_ref,
                mean_ref, lv_ref, xhat_ref,
                xs_s, acc_s, hd_s):
    s = pl.program_id(0)

    @pl.when(s < GS)
    def _gcn():
        a16 = a_ref[...]
        lo = (a16 & 0xFFFF).astype(jnp.float32)
        hi = ((a16 >> 16) & 0xFFFF).astype(jnp.float32)
        xg = x_ref[...]
        h = lax.dot_general(xg, w_ref[...], (((2,), (0,)), ((), ())),
                            preferred_element_type=jnp.float32)
        deg = jnp.sum(lo, axis=2) + jnp.sum(hi, axis=2) + 1.0
        dinv = lax.rsqrt(deg)
        hs = h * dinv[:, :, None]
        hs2 = hs.reshape(GB, N // 2, 2, D)
        hs_even = hs2[:, :, 0, :]
        hs_odd = hs2[:, :, 1, :]
        bd = (((2,), (1,)), ((0,), (0,)))
        out = (lax.dot_general(lo, hs_even, bd,
                               preferred_element_type=jnp.float32)
               + lax.dot_general(hi, hs_odd, bd,
                                 preferred_element_type=jnp.float32)
               + hs)
        out = out * dinv[:, :, None] + b_ref[...]
        mu = jnp.mean(out, axis=1, keepdims=True)
        var = jnp.mean((out - mu) * (out - mu), axis=1, keepdims=True)
        out = (out - mu) * lax.rsqrt(var + 1e-5) * gam_ref[...] + bet_ref[...]
        xs_s[pl.ds(s * GB, GB), :] = (
            jnp.tanh(out).astype(jnp.bfloat16).reshape(GB, N * D))

    @pl.when(s == GS)
    def _init():
        acc_s[...] = jnp.zeros_like(acc_s)

    @pl.when((s >= GS) & (s < GS + ES))
    def _enc():
        k = s - GS
        xs_blk = xs_s[:, pl.ds(k * KC, KC)]
        acc_s[...] += jnp.dot(xs_blk, w1_ref[...],
                              preferred_element_type=jnp.float32)

    @pl.when(s == GS + ES - 1)
    def _head():
        h1 = jnp.maximum(acc_s[...] + b1_ref[...], 0.0)
        mean = jnp.dot(h1, wmu_ref[...],
                       preferred_element_type=jnp.float32) + bmu_ref[...]
        lv = jnp.dot(h1, wlv_ref[...],
                     preferred_element_type=jnp.float32) + blv_ref[...]
        z = mean + jnp.exp(0.5 * lv) * eps_ref[...]
        hd = jnp.dot(z, dw1_ref[...],
                     preferred_element_type=jnp.float32) + db1_ref[...]
        mean_ref[...] = mean
        lv_ref[...] = lv
        hd_s[...] = jnp.maximum(hd, 0.0).astype(jnp.bfloat16)

    @pl.when(s >= GS + ES)
    def _dec():
        xhat_ref[...] = jax.nn.sigmoid(
            jnp.dot(hd_s[...], w2_ref[...],
                    preferred_element_type=jnp.float32) + b2_ref[...])


def _fused(A3, x, gcn_W, gcn_b, bn_gamma, bn_beta,
           enc_W1, enc_b1, enc_Wmu, enc_bmu, enc_Wlv, enc_blv,
           dec_W1, dec_b1, eps, dec_W2, dec_b2):
    nsteps = GS + ES + DS
    c0 = lambda s: (0, 0)
    c000 = lambda s: (0, 0, 0)
    return pl.pallas_call(
        _fused_body,
        grid=(nsteps,),
        in_specs=[
            pl.BlockSpec((GB, N, N // 2),
                         lambda s: (jnp.minimum(s, GS - 1), 0, 0)),
            pl.BlockSpec((GB, N, D), lambda s: (jnp.minimum(s, GS - 1), 0, 0)),
            pl.BlockSpec((D, D), c0),
            pl.BlockSpec((1, 1, D), c000),
            pl.BlockSpec((1, 1, D), c000),
            pl.BlockSpec((1, 1, D), c000),
            pl.BlockSpec((KC, HIDDEN),
                         lambda s: (jnp.clip(s - GS, 0, ES - 1), 0)),
            pl.BlockSpec((1, HIDDEN), c0),
            pl.BlockSpec((HIDDEN, L), c0),
            pl.BlockSpec((1, L), c0),
            pl.BlockSpec((HIDDEN, L), c0),
            pl.BlockSpec((1, L), c0),
            pl.BlockSpec((L, HIDDEN), c0),
            pl.BlockSpec((1, HIDDEN), c0),
            pl.BlockSpec((B, L), c0),
            pl.BlockSpec((HIDDEN, NC),
                         lambda s: (0, jnp.clip(s - GS - ES, 0, DS - 1))),
            pl.BlockSpec((1, NC),
                         lambda s: (0, jnp.clip(s - GS - ES, 0, DS - 1))),
        ],
        out_specs=[
            pl.BlockSpec((B, L), c0),
            pl.BlockSpec((B, L), c0),
            pl.BlockSpec((B, NC),
                         lambda s: (0, jnp.clip(s - GS - ES, 0, DS - 1))),
        ],
        out_shape=[
            jax.ShapeDtypeStruct((B, L), jnp.float32),
            jax.ShapeDtypeStruct((B, L), jnp.float32),
            jax.ShapeDtypeStruct((B, INPUT_DIM), jnp.float32),
        ],
        scratch_shapes=[
            pltpu.VMEM((B, INPUT_DIM), jnp.bfloat16),
            pltpu.VMEM((B, HIDDEN), jnp.float32),
            pltpu.VMEM((B, HIDDEN), jnp.bfloat16),
        ],
        compiler_params=pltpu.CompilerParams(
            dimension_semantics=("arbitrary",),
        ),
    )(A3, x, gcn_W, gcn_b.reshape(1, 1, D), bn_gamma.reshape(1, 1, D),
      bn_beta.reshape(1, 1, D), enc_W1, enc_b1.reshape(1, HIDDEN),
      enc_Wmu, enc_bmu.reshape(1, L), enc_Wlv, enc_blv.reshape(1, L),
      dec_W1, dec_b1.reshape(1, HIDDEN), eps, dec_W2,
      dec_b2.reshape(1, INPUT_DIM))


def kernel(x, edge_index, gcn_W, gcn_b, bn_gamma, bn_beta,
           enc_W1, enc_b1, enc_Wmu, enc_bmu, enc_Wlv, enc_blv,
           dec_W1, dec_b1, dec_W2, dec_b2, eps):
    A3 = _build_adj_fn()(edge_index)
    mean, log_var, x_hat = _fused(
        A3, x, gcn_W, gcn_b, bn_gamma, bn_beta,
        enc_W1, enc_b1, enc_Wmu, enc_bmu, enc_Wlv, enc_blv,
        dec_W1, dec_b1, eps, dec_W2, dec_b2)
    return (x_hat, mean, log_var)
